# baseline (speedup 1.0000x reference)
import functools

import jax
import jax.numpy as jnp
from jax import lax
from jax.experimental import pallas as pl
from jax.experimental.pallas import tpu as pltpu
from jax.experimental.pallas import tpu_sc as plsc

N = 10000
E = 320000
D = 128

NC = 2
NS = 16
NW = NC * NS

B_EPT = E // NW
B_K = 125
B_C = B_EPT // B_K
B_H = B_C // 2
B_S = 16

G_EPT = E // NS
G_K = 100
G_C = G_EPT // G_K
G_S = 40

NPAD = 10240
NPT = NPAD // NS
SCH = 40

_mesh = plsc.VectorSubcoreMesh(core_axis_name="c", subcore_axis_name="s")


@functools.partial(
    pl.kernel,
    mesh=_mesh,
    out_type=[
        jax.ShapeDtypeStruct((NPAD,), jnp.float32),
        jax.ShapeDtypeStruct((NPAD, D), jnp.float32),
        jax.ShapeDtypeStruct((NC, NPAD, D), jnp.float32),
    ],
    scratch_types=[
        pltpu.VMEM((112,), jnp.float32),
        pltpu.VMEM((G_S, G_K), jnp.int32),
        pltpu.VMEM((NPT,), jnp.float32),
        pltpu.VMEM((B_S, B_K), jnp.int32),
        pltpu.VMEM((B_S, B_K), jnp.int32),
        pltpu.VMEM((2, B_K, D), jnp.float32),
        pltpu.VMEM_SHARED((NPAD,), jnp.float32),
        pltpu.VMEM_SHARED((NPAD, D), jnp.float32),
        pltpu.SemaphoreType.DMA,
        pltpu.SemaphoreType.DMA,
    ],
)
def _sca_kernel(x_hbm, srcg_hbm, srcb_hbm, dstb_hbm, dinv_hbm, xs_hbm, p_hbm,
                ones_v, didx_v, dbuf_v, src_v, dst_v, gbuf_v,
                hist_sh, acc_sh, dsem, gsem):
    c = lax.axis_index("c")
    s = lax.axis_index("s")

    for i in range(7):
        ones_v[pl.ds(i * 16, 16)] = jnp.full((16,), 1.0, jnp.float32)

    def _zero(i, carry):
        dbuf_v[pl.ds(i * 16, 16)] = jnp.zeros((16,), jnp.float32)
        return carry

    lax.fori_loop(0, NPT // 16, _zero, 0)
    pltpu.sync_copy(dbuf_v, hist_sh.at[pl.ds(s * NPT, NPT)])

    def _zrow(r, carry):
        for q in range(D // 16):
            gbuf_v[0, r, pl.ds(q * 16, 16)] = jnp.zeros((16,), jnp.float32)
        return carry

    lax.fori_loop(0, B_K, _zrow, 0)
    for t in range(NPT // B_K):
        pltpu.sync_copy(gbuf_v.at[0], acc_sh.at[pl.ds(s * NPT + t * B_K, B_K)])
    pltpu.sync_copy(gbuf_v.at[0, pl.ds(0, NPT % B_K)],
                    acc_sh.at[pl.ds(s * NPT + (NPT // B_K) * B_K, NPT % B_K)])
    plsc.subcore_barrier()

    def _hd(j):
        return pltpu.make_async_copy(
            ones_v.at[pl.ds(0, G_K)], hist_sh.at[didx_v.at[j]], dsem)

    GG = 8
    NG = G_S // GG

    def _fire(base):
        for u in range(GG):
            _hd(base + u).start(add=True)

    def _drain(base):
        for u in range(GG):
            _hd(base + u).wait()

    for st in range(G_C // G_S):
        pltpu.sync_copy(srcg_hbm.at[s, pl.ds(st * G_S, G_S)], didx_v)
        _fire(0)

        def _hist(g, carry):
            _fire((g + 1) * GG)
            _drain(g * GG)
            return carry

        lax.fori_loop(0, NG - 1, _hist, 0)
        _drain((NG - 1) * GG)
    plsc.subcore_barrier()

    pltpu.sync_copy(hist_sh.at[pl.ds(s * NPT, NPT)], dbuf_v)

    def _recip(i, carry):
        v = dbuf_v[pl.ds(i * 16, 16)]
        dbuf_v[pl.ds(i * 16, 16)] = 1.0 / jnp.maximum(v, 1.0)
        return carry

    lax.fori_loop(0, NPT // 16, _recip, 0)

    @pl.when(c == 0)
    def _():
        pltpu.sync_copy(dbuf_v, dinv_hbm.at[pl.ds(s * NPT, NPT)])

    def _schunk(k, carry):
        c0 = s * NPT + k * 16

        @pl.when(c0 < N)
        def _():
            pltpu.sync_copy(x_hbm.at[pl.ds(c0, 16)], gbuf_v.at[0, pl.ds(0, 16)])
            dv16 = dbuf_v[pl.ds(k * 16, 16)]
            for i in range(16):
                dv = lax.gather(
                    dv16,
                    jnp.full((16, 1), i, jnp.int32),
                    lax.GatherDimensionNumbers(
                        offset_dims=(), collapsed_slice_dims=(0,),
                        start_index_map=(0,)),
                    (1,),
                    mode=lax.GatherScatterMode.PROMISE_IN_BOUNDS,
                )
                for q in range(D // 16):
                    gbuf_v[0, i, pl.ds(q * 16, 16)] = (
                        gbuf_v[0, i, pl.ds(q * 16, 16)] * dv)
            pltpu.sync_copy(gbuf_v.at[0, pl.ds(0, 16)], xs_hbm.at[pl.ds(c0, 16)])

        return carry

    lax.fori_loop(0, NPT // 16, _schunk, 0)
    plsc.subcore_barrier()

    def _gd(j, b):
        return pltpu.make_async_copy(xs_hbm.at[src_v.at[j]], gbuf_v.at[b], gsem)

    for stg in range(B_C // B_S):
        pltpu.sync_copy(srcb_hbm.at[c, s, pl.ds(stg * B_S, B_S)], src_v)
        pltpu.sync_copy(dstb_hbm.at[c, s, pl.ds(stg * B_S, B_S)], dst_v)

        _gd(0, 0).start()

        def _body(jo, carry):
            for b in range(2):
                j = jo * 2 + b
                if b == 0:
                    _gd(j + 1, 1).start()
                else:
                    @pl.when(jo != B_S // 2 - 1)
                    def _():
                        _gd(j + 1, 0).start()
                _gd(j, b).wait()
                pltpu.sync_copy(gbuf_v.at[b], acc_sh.at[dst_v.at[j]], add=True)
            return carry

        lax.fori_loop(0, B_S // 2, _body, 0)

    plsc.subcore_barrier()

    pltpu.sync_copy(acc_sh.at[pl.ds(s * NPT, NPT)], p_hbm.at[c, pl.ds(s * NPT, NPT)])


@functools.partial(
    pl.kernel,
    mesh=_mesh,
    out_type=jax.ShapeDtypeStruct((NC, NPAD, D), jnp.float32),
    scratch_types=[
        pltpu.VMEM((B_H, B_K), jnp.int32),
        pltpu.VMEM((B_H, B_K), jnp.int32),
        pltpu.VMEM((2, B_K, D), jnp.float32),
        pltpu.VMEM_SHARED((NPAD, D), jnp.float32),
        pltpu.SemaphoreType.DMA,
    ],
)
def _agg_kernel(h_hbm, src_hbm, dst_hbm, out_hbm, src_v, dst_v, gbuf_v, acc_sh, gsem):
    c = lax.axis_index("c")
    s = lax.axis_index("s")

    def _zrow(r, carry):
        for q in range(D // 16):
            gbuf_v[0, r, pl.ds(q * 16, 16)] = jnp.zeros((16,), jnp.float32)
        return carry

    lax.fori_loop(0, B_K, _zrow, 0)
    for t in range(NPT // B_K):
        pltpu.sync_copy(gbuf_v.at[0], acc_sh.at[pl.ds(s * NPT + t * B_K, B_K)])
    pltpu.sync_copy(gbuf_v.at[0, pl.ds(0, NPT % B_K)],
                    acc_sh.at[pl.ds(s * NPT + (NPT // B_K) * B_K, NPT % B_K)])
    plsc.subcore_barrier()

    def _gd(j, b):
        return pltpu.make_async_copy(h_hbm.at[src_v.at[j]], gbuf_v.at[b], gsem)

    for half in range(2):
        pltpu.sync_copy(src_hbm.at[c, s, pl.ds(half * B_H, B_H)], src_v)
        pltpu.sync_copy(dst_hbm.at[c, s, pl.ds(half * B_H, B_H)], dst_v)

        _gd(0, 0).start()

        def _body(jo, carry):
            for b in range(2):
                j = jo * 2 + b
                if b == 0:
                    _gd(j + 1, 1).start()
                else:
                    @pl.when(jo != B_H // 2 - 1)
                    def _():
                        _gd(j + 1, 0).start()
                _gd(j, b).wait()
                pltpu.sync_copy(gbuf_v.at[b], acc_sh.at[dst_v.at[j]], add=True)
            return carry

        lax.fori_loop(0, B_H // 2, _body, 0)

    plsc.subcore_barrier()

    pltpu.sync_copy(acc_sh.at[pl.ds(s * NPT, NPT)], out_hbm.at[c, pl.ds(s * NPT, NPT)])


R = 2000


def _mid_body(p0_ref, p1_ref, b_ref, w_ref, d_ref, o_ref):
    h = lax.dot_general(
        p0_ref[...] + p1_ref[...], w_ref[...], (((1,), (1,)), ((), ())),
        precision=lax.Precision.HIGHEST,
        preferred_element_type=jnp.float32,
    )
    o_ref[...] = jnp.maximum(h + b_ref[...], 0.0) * d_ref[...]


_mid = pl.pallas_call(
    _mid_body,
    grid=(N // R,),
    in_specs=[
        pl.BlockSpec((R, D), lambda i: (i, 0)),
        pl.BlockSpec((R, D), lambda i: (i, 0)),
        pl.BlockSpec((1, D), lambda i: (0, 0)),
        pl.BlockSpec((D, D), lambda i: (0, 0)),
        pl.BlockSpec((R, 1), lambda i: (i, 0)),
    ],
    out_specs=pl.BlockSpec((R, D), lambda i: (i, 0)),
    out_shape=jax.ShapeDtypeStruct((N, D), jnp.float32),
)


def _out_body(q0_ref, q1_ref, b_ref, w_ref, o_ref):
    h = lax.dot_general(
        q0_ref[...] + q1_ref[...], w_ref[...], (((1,), (1,)), ((), ())),
        precision=lax.Precision.HIGHEST,
        preferred_element_type=jnp.float32,
    )
    o_ref[...] = h + b_ref[...]


_out = pl.pallas_call(
    _out_body,
    grid=(N // R,),
    in_specs=[
        pl.BlockSpec((R, D), lambda i: (i, 0)),
        pl.BlockSpec((R, D), lambda i: (i, 0)),
        pl.BlockSpec((1, D), lambda i: (0, 0)),
        pl.BlockSpec((D, D), lambda i: (0, 0)),
    ],
    out_specs=pl.BlockSpec((R, D), lambda i: (i, 0)),
    out_shape=jax.ShapeDtypeStruct((N, D), jnp.float32),
)


def kernel(x, edge_index, W1, b1, W2, b2):
    ei = edge_index.astype(jnp.int32)
    src, dst = ei[0], ei[1]
    src_b = src.reshape(NC, NS, B_C, B_K)
    dst_b = dst.reshape(NC, NS, B_C, B_K)
    src_g = src.reshape(NS, G_C, G_K)

    dinv, xs, p = _sca_kernel(x, src_g, src_b, dst_b)
    del xs
    dcol = dinv[:N].reshape(N, 1)

    hs = _mid(p[0, :N], p[1, :N], b1.reshape(1, D), W1, dcol)
    q = _agg_kernel(hs, src_b, dst_b)
    return _out(q[0, :N], q[1, :N], b2.reshape(1, D), W2)


# speedup vs baseline: 1.1559x; 1.1559x over previous
import functools

import jax
import jax.numpy as jnp
from jax import lax
from jax.experimental import pallas as pl
from jax.experimental.pallas import tpu as pltpu
from jax.experimental.pallas import tpu_sc as plsc

N = 10000
E = 320000
D = 128

NC = 2
NS = 16
NW = NC * NS

B_EPT = E // NW
B_K = 125
B_C = B_EPT // B_K
B_H = B_C // 2
B_S = 16

G_K = 100
G_C = B_EPT // G_K
GG = 5

NPAD = 10240
NPT = NPAD // NS
SCH = 40

_mesh = plsc.VectorSubcoreMesh(core_axis_name="c", subcore_axis_name="s")


@functools.partial(
    pl.kernel,
    mesh=_mesh,
    out_type=jax.ShapeDtypeStruct((NC, NPAD), jnp.float32),
    scratch_types=[
        pltpu.VMEM((112,), jnp.float32),
        pltpu.VMEM((G_C, G_K), jnp.int32),
        pltpu.VMEM((NPT,), jnp.float32),
        pltpu.VMEM_SHARED((NPAD,), jnp.float32),
        pltpu.SemaphoreType.DMA,
    ],
)
def _deg_kernel(src_hbm, hist_hbm, ones_v, didx_v, zbuf_v, hist_sh, dsem):
    c = lax.axis_index("c")
    s = lax.axis_index("s")

    for i in range(7):
        ones_v[pl.ds(i * 16, 16)] = jnp.full((16,), 1.0, jnp.float32)

    def _zero(i, carry):
        zbuf_v[pl.ds(i * 16, 16)] = jnp.zeros((16,), jnp.float32)
        return carry

    lax.fori_loop(0, NPT // 16, _zero, 0)
    pltpu.sync_copy(zbuf_v, hist_sh.at[pl.ds(s * NPT, NPT)])
    plsc.subcore_barrier()

    pltpu.sync_copy(src_hbm.at[c, s], didx_v)

    def _hd(j):
        return pltpu.make_async_copy(
            ones_v.at[pl.ds(0, G_K)], hist_sh.at[didx_v.at[j]], dsem)

    def _fire(base):
        for u in range(GG):
            _hd(base + u).start(add=True)

    def _drain(base):
        for u in range(GG):
            _hd(base + u).wait()

    _fire(0)

    def _hist(g, carry):
        _fire((g + 1) * GG)
        _drain(g * GG)
        return carry

    lax.fori_loop(0, G_C // GG - 1, _hist, 0)
    _drain(G_C - GG)
    plsc.subcore_barrier()

    pltpu.sync_copy(hist_sh.at[pl.ds(s * NPT, NPT)],
                    hist_hbm.at[c, pl.ds(s * NPT, NPT)])


@functools.partial(
    pl.kernel,
    mesh=_mesh,
    out_type=jax.ShapeDtypeStruct((NC, NPAD, D), jnp.float32),
    scratch_types=[
        pltpu.VMEM((B_H, B_K), jnp.int32),
        pltpu.VMEM((B_H, B_K), jnp.int32),
        pltpu.VMEM((2, B_K, D), jnp.float32),
        pltpu.VMEM_SHARED((NPAD, D), jnp.float32),
        pltpu.SemaphoreType.DMA,
    ],
)
def _agg_kernel(h_hbm, src_hbm, dst_hbm, out_hbm, src_v, dst_v, gbuf_v, acc_sh, gsem):
    c = lax.axis_index("c")
    s = lax.axis_index("s")

    def _zrow(r, carry):
        for q in range(D // 16):
            gbuf_v[0, r, pl.ds(q * 16, 16)] = jnp.zeros((16,), jnp.float32)
        return carry

    lax.fori_loop(0, B_K, _zrow, 0)
    for t in range(NPT // B_K):
        pltpu.sync_copy(gbuf_v.at[0], acc_sh.at[pl.ds(s * NPT + t * B_K, B_K)])
    pltpu.sync_copy(gbuf_v.at[0, pl.ds(0, NPT % B_K)],
                    acc_sh.at[pl.ds(s * NPT + (NPT // B_K) * B_K, NPT % B_K)])
    plsc.subcore_barrier()

    def _gd(j, b):
        return pltpu.make_async_copy(h_hbm.at[src_v.at[j]], gbuf_v.at[b], gsem)

    for half in range(2):
        pltpu.sync_copy(src_hbm.at[c, s, pl.ds(half * B_H, B_H)], src_v)
        pltpu.sync_copy(dst_hbm.at[c, s, pl.ds(half * B_H, B_H)], dst_v)

        _gd(0, 0).start()

        def _body(jo, carry):
            for b in range(2):
                j = jo * 2 + b
                if b == 0:
                    _gd(j + 1, 1).start()
                else:
                    @pl.when(jo != B_H // 2 - 1)
                    def _():
                        _gd(j + 1, 0).start()
                _gd(j, b).wait()
                pltpu.sync_copy(gbuf_v.at[b], acc_sh.at[dst_v.at[j]], add=True)
            return carry

        lax.fori_loop(0, B_H // 2, _body, 0)

    plsc.subcore_barrier()

    pltpu.sync_copy(acc_sh.at[pl.ds(s * NPT, NPT)], out_hbm.at[c, pl.ds(s * NPT, NPT)])


R = 2000


def _scale_body(h0_ref, h1_ref, x_ref, xs_ref, d_ref):
    dinv = 1.0 / jnp.maximum(h0_ref[...] + h1_ref[...], 1.0)
    xs_ref[...] = x_ref[...] * dinv
    d_ref[...] = dinv


_scale = pl.pallas_call(
    _scale_body,
    grid=(N // R,),
    in_specs=[
        pl.BlockSpec((R, 1), lambda i: (i, 0)),
        pl.BlockSpec((R, 1), lambda i: (i, 0)),
        pl.BlockSpec((R, D), lambda i: (i, 0)),
    ],
    out_specs=[
        pl.BlockSpec((R, D), lambda i: (i, 0)),
        pl.BlockSpec((R, 1), lambda i: (i, 0)),
    ],
    out_shape=[
        jax.ShapeDtypeStruct((N, D), jnp.float32),
        jax.ShapeDtypeStruct((N, 1), jnp.float32),
    ],
)


def _mid_body(p0_ref, p1_ref, b_ref, w_ref, d_ref, o_ref):
    h = lax.dot_general(
        p0_ref[...] + p1_ref[...], w_ref[...], (((1,), (1,)), ((), ())),
        precision=lax.Precision.HIGHEST,
        preferred_element_type=jnp.float32,
    )
    o_ref[...] = jnp.maximum(h + b_ref[...], 0.0) * d_ref[...]


_mid = pl.pallas_call(
    _mid_body,
    grid=(N // R,),
    in_specs=[
        pl.BlockSpec((R, D), lambda i: (i, 0)),
        pl.BlockSpec((R, D), lambda i: (i, 0)),
        pl.BlockSpec((1, D), lambda i: (0, 0)),
        pl.BlockSpec((D, D), lambda i: (0, 0)),
        pl.BlockSpec((R, 1), lambda i: (i, 0)),
    ],
    out_specs=pl.BlockSpec((R, D), lambda i: (i, 0)),
    out_shape=jax.ShapeDtypeStruct((N, D), jnp.float32),
)


def _out_body(q0_ref, q1_ref, b_ref, w_ref, o_ref):
    h = lax.dot_general(
        q0_ref[...] + q1_ref[...], w_ref[...], (((1,), (1,)), ((), ())),
        precision=lax.Precision.HIGHEST,
        preferred_element_type=jnp.float32,
    )
    o_ref[...] = h + b_ref[...]


_out = pl.pallas_call(
    _out_body,
    grid=(N // R,),
    in_specs=[
        pl.BlockSpec((R, D), lambda i: (i, 0)),
        pl.BlockSpec((R, D), lambda i: (i, 0)),
        pl.BlockSpec((1, D), lambda i: (0, 0)),
        pl.BlockSpec((D, D), lambda i: (0, 0)),
    ],
    out_specs=pl.BlockSpec((R, D), lambda i: (i, 0)),
    out_shape=jax.ShapeDtypeStruct((N, D), jnp.float32),
)


def kernel(x, edge_index, W1, b1, W2, b2):
    ei = edge_index.astype(jnp.int32)
    src, dst = ei[0], ei[1]
    src_b = src.reshape(NC, NS, B_C, B_K)
    dst_b = dst.reshape(NC, NS, B_C, B_K)
    src_g = src.reshape(NC, NS, G_C, G_K)

    hist = _deg_kernel(src_g)
    h0 = hist[0, :N].reshape(N, 1)
    h1 = hist[1, :N].reshape(N, 1)
    xs, dcol = _scale(h0, h1, x)
    p = _agg_kernel(xs, src_b, dst_b)
    hs = _mid(p[0, :N], p[1, :N], b1.reshape(1, D), W1, dcol)
    q = _agg_kernel(hs, src_b, dst_b)
    return _out(q[0, :N], q[1, :N], b2.reshape(1, D), W2)
